# Initial kernel scaffold; baseline (speedup 1.0000x reference)
import jax
import jax.numpy as jnp
from jax.experimental import pallas as pl


def kernel(soutput_f_l, soutput_f_r, matches, non_matches, start_idx, num_points, W1, b1, W2, b2):
    raise NotImplementedError("write your pallas kernel here")


# speedup vs baseline: 3.1422x; 3.1422x over previous
import functools

import jax
import jax.numpy as jnp
from jax import lax
from jax.experimental import pallas as pl
from jax.experimental.pallas import tpu as pltpu
from jax.experimental.pallas import tpu_sc as plsc


def _fused_gather(fl, fr, il, ir):
    n_rows = il.shape[0] * 128
    D = fl.shape[1]
    info = plsc.get_sparse_core_info()
    NC, NS = info.num_cores, info.num_subcores
    NW = NC * NS
    rows_per_w = n_rows // NW
    CHUNK = 512
    n_chunk = rows_per_w // CHUNK
    JPC = CHUNK // 128
    idx_rows_w = rows_per_w // 128

    mesh = plsc.VectorSubcoreMesh(core_axis_name="c", subcore_axis_name="s")

    @functools.partial(
        pl.kernel,
        mesh=mesh,
        compiler_params=pltpu.CompilerParams(use_tc_tiling_on_sc=False),
        out_type=(
            jax.ShapeDtypeStruct((n_rows, D), jnp.float32),
            jax.ShapeDtypeStruct((n_rows, D), jnp.float32),
        ),
        scratch_types=[
            pltpu.VMEM((idx_rows_w, 128), jnp.int32),
            pltpu.VMEM((idx_rows_w, 128), jnp.int32),
            pltpu.VMEM((CHUNK, D), jnp.float32),
            pltpu.VMEM((CHUNK, D), jnp.float32),
            pltpu.SemaphoreType.DMA,
            pltpu.SemaphoreType.DMA,
        ],
    )
    def gather_kernel(fl_hbm, fr_hbm, il_hbm, ir_hbm, ol_hbm, or_hbm,
                      il_v, ir_v, rl_v, rr_v, sem_l, sem_r):
        wid = lax.axis_index("s") * NC + lax.axis_index("c")
        pltpu.sync_copy(il_hbm.at[pl.ds(wid * idx_rows_w, idx_rows_w)], il_v)
        pltpu.sync_copy(ir_hbm.at[pl.ds(wid * idx_rows_w, idx_rows_w)], ir_v)

        def chunk_body(ci, carry):
            cps = []
            for j in range(JPC):
                row = ci * JPC + j
                cps.append(pltpu.async_copy(
                    fl_hbm.at[il_v.at[row]], rl_v.at[pl.ds(j * 128, 128)],
                    sem_l))
                cps.append(pltpu.async_copy(
                    fr_hbm.at[ir_v.at[row]], rr_v.at[pl.ds(j * 128, 128)],
                    sem_r))
            for cp in cps:
                cp.wait()
            base = wid * rows_per_w + ci * CHUNK
            pltpu.sync_copy(rl_v, ol_hbm.at[pl.ds(base, CHUNK)])
            pltpu.sync_copy(rr_v, or_hbm.at[pl.ds(base, CHUNK)])
            return carry

        lax.fori_loop(0, n_chunk, chunk_body, 0)

    return gather_kernel(fl, fr, il, ir)


def _mlp(fl, fr, w1t, w1b, b1, w2, b2):
    n_rows, D = fl.shape
    H = w1t.shape[1]
    O = w2.shape[1]
    TM = 2048
    grid = (n_rows // TM,)

    def body(fl_ref, fr_ref, w1t_ref, w1b_ref, b1_ref, w2_ref, b2_ref,
             out_ref):
        h = jnp.dot(fl_ref[...], w1t_ref[...],
                    preferred_element_type=jnp.float32)
        h = h + jnp.dot(fr_ref[...], w1b_ref[...],
                        preferred_element_type=jnp.float32)
        h = jnp.maximum(h + b1_ref[...], 0.0)
        out_ref[...] = jnp.dot(h, w2_ref[...],
                               preferred_element_type=jnp.float32) + b2_ref[...]

    return pl.pallas_call(
        body,
        grid=grid,
        in_specs=[
            pl.BlockSpec((TM, D), lambda i: (i, 0)),
            pl.BlockSpec((TM, D), lambda i: (i, 0)),
            pl.BlockSpec((D, H), lambda i: (0, 0)),
            pl.BlockSpec((D, H), lambda i: (0, 0)),
            pl.BlockSpec((1, H), lambda i: (0, 0)),
            pl.BlockSpec((H, O), lambda i: (0, 0)),
            pl.BlockSpec((1, O), lambda i: (0, 0)),
        ],
        out_specs=pl.BlockSpec((TM, O), lambda i: (i, 0)),
        out_shape=jax.ShapeDtypeStruct((n_rows, O), jnp.float32),
    )(fl, fr, w1t, w1b, b1, w2, b2)


def kernel(soutput_f_l, soutput_f_r, matches, non_matches, start_idx,
           num_points, W1, b1, W2, b2):
    B, M, _ = matches.shape
    NM = non_matches.shape[1]
    R, D = soutput_f_l.shape
    n_static = R // B

    def global_idx(x):
        x = jnp.where(x > -1, x, num_points[:, None])
        x = jnp.clip(x, 0, n_static - 1)
        return x + start_idx[:, None]

    il = jnp.concatenate(
        [global_idx(matches[:, :, 0]), global_idx(non_matches[:, :, 0])],
        axis=0)
    ir = jnp.concatenate(
        [global_idx(matches[:, :, 1]), global_idx(non_matches[:, :, 1])],
        axis=0)
    n_rows = B * (M + NM)
    il = il.reshape(n_rows // 128, 128)
    ir = ir.reshape(n_rows // 128, 128)

    fl_g, fr_g = _fused_gather(soutput_f_l, soutput_f_r, il, ir)

    prediction = _mlp(fl_g, fr_g, W1[:D], W1[D:], b1.reshape(1, -1), W2,
                      b2.reshape(1, -1))
    label = jnp.concatenate(
        [jnp.ones((B * M,), jnp.float32), jnp.zeros((B * NM,), jnp.float32)])
    return (prediction, label)


# speedup vs baseline: 3.6659x; 1.1667x over previous
import functools

import jax
import jax.numpy as jnp
from jax import lax
from jax.experimental import pallas as pl
from jax.experimental.pallas import tpu as pltpu
from jax.experimental.pallas import tpu_sc as plsc


def _fused_gather(tables, il, ir):
    n_rows = il.shape[0] * 128
    D = tables.shape[2]
    info = plsc.get_sparse_core_info()
    NC, NS = info.num_cores, info.num_subcores
    NW = NC * NS
    rows_per_w = n_rows // NW
    CHUNK = 256
    n_chunk = rows_per_w // CHUNK
    JPC = CHUNK // 128
    idx_rows_w = rows_per_w // 128

    mesh = plsc.VectorSubcoreMesh(core_axis_name="c", subcore_axis_name="s")

    @functools.partial(
        pl.kernel,
        mesh=mesh,
        compiler_params=pltpu.CompilerParams(use_tc_tiling_on_sc=False),
        out_type=jax.ShapeDtypeStruct((n_rows, 2 * D), jnp.float32),
        scratch_types=[
            pltpu.VMEM((idx_rows_w, 128), jnp.int32),
            pltpu.VMEM((idx_rows_w, 128), jnp.int32),
            pltpu.VMEM((CHUNK, D), jnp.float32),
            pltpu.VMEM((CHUNK, D), jnp.float32),
            pltpu.SemaphoreType.DMA,
            pltpu.SemaphoreType.DMA,
        ],
    )
    def gather_kernel(t_hbm, il_hbm, ir_hbm, o_hbm,
                      il_v, ir_v, rl_v, rr_v, sem_l, sem_r):
        wid = lax.axis_index("s") * NC + lax.axis_index("c")
        pltpu.sync_copy(il_hbm.at[pl.ds(wid * idx_rows_w, idx_rows_w)], il_v)
        pltpu.sync_copy(ir_hbm.at[pl.ds(wid * idx_rows_w, idx_rows_w)], ir_v)
        fl_hbm = t_hbm.at[0]
        fr_hbm = t_hbm.at[1]

        def chunk_body(ci, carry):
            cps = []
            for j in range(JPC):
                row = ci * JPC + j
                cps.append(pltpu.async_copy(
                    fl_hbm.at[il_v.at[row]], rl_v.at[pl.ds(j * 128, 128)],
                    sem_l))
                cps.append(pltpu.async_copy(
                    fr_hbm.at[ir_v.at[row]], rr_v.at[pl.ds(j * 128, 128)],
                    sem_r))
            for cp in cps:
                cp.wait()
            base = wid * rows_per_w + ci * CHUNK
            pltpu.sync_copy(rl_v,
                            o_hbm.at[pl.ds(base, CHUNK), pl.ds(0, D)])
            pltpu.sync_copy(rr_v,
                            o_hbm.at[pl.ds(base, CHUNK), pl.ds(D, D)])
            return carry

        lax.fori_loop(0, n_chunk, chunk_body, 0)

    return gather_kernel(tables, il, ir)


def _mlp(f, w1, b1, w2, b2):
    n_rows, D2 = f.shape
    H = w1.shape[1]
    O = w2.shape[1]
    TM = 2048
    grid = (n_rows // TM,)

    def body(f_ref, w1_ref, b1_ref, w2_ref, b2_ref, out_ref):
        h = jnp.dot(f_ref[...], w1_ref[...],
                    preferred_element_type=jnp.float32)
        h = jnp.maximum(h + b1_ref[...], 0.0)
        out_ref[...] = jnp.dot(h, w2_ref[...],
                               preferred_element_type=jnp.float32) + b2_ref[...]

    return pl.pallas_call(
        body,
        grid=grid,
        in_specs=[
            pl.BlockSpec((TM, D2), lambda i: (i, 0)),
            pl.BlockSpec((D2, H), lambda i: (0, 0)),
            pl.BlockSpec((1, H), lambda i: (0, 0)),
            pl.BlockSpec((H, O), lambda i: (0, 0)),
            pl.BlockSpec((1, O), lambda i: (0, 0)),
        ],
        out_specs=pl.BlockSpec((TM, O), lambda i: (i, 0)),
        out_shape=jax.ShapeDtypeStruct((n_rows, O), jnp.float32),
    )(f, w1, b1, w2, b2)


def kernel(soutput_f_l, soutput_f_r, matches, non_matches, start_idx,
           num_points, W1, b1, W2, b2):
    B, M, _ = matches.shape
    NM = non_matches.shape[1]
    R, D = soutput_f_l.shape
    n_static = R // B

    def global_idx(x):
        x = jnp.where(x > -1, x, num_points[:, None])
        x = jnp.clip(x, 0, n_static - 1)
        return x + start_idx[:, None]

    il = jnp.concatenate(
        [global_idx(matches[:, :, 0]), global_idx(non_matches[:, :, 0])],
        axis=0)
    ir = jnp.concatenate(
        [global_idx(matches[:, :, 1]), global_idx(non_matches[:, :, 1])],
        axis=0)
    n_rows = B * (M + NM)
    il = il.reshape(n_rows // 128, 128)
    ir = ir.reshape(n_rows // 128, 128)

    tables = jnp.stack([soutput_f_l, soutput_f_r])
    f = _fused_gather(tables, il, ir)

    prediction = _mlp(f, W1, b1.reshape(1, -1), W2, b2.reshape(1, -1))
    label = jnp.concatenate(
        [jnp.ones((B * M,), jnp.float32), jnp.zeros((B * NM,), jnp.float32)])
    return (prediction, label)


# speedup vs baseline: 4.8188x; 1.3145x over previous
import functools

import jax
import jax.numpy as jnp
from jax import lax
from jax.experimental import pallas as pl
from jax.experimental.pallas import tpu as pltpu
from jax.experimental.pallas import tpu_sc as plsc


def _fused_gather(table, il, ir):
    n_rows = il.shape[0] * 128
    D = table.shape[1]
    info = plsc.get_sparse_core_info()
    NC, NS = info.num_cores, info.num_subcores
    NW = NC * NS
    rows_per_w = n_rows // NW
    CHUNK = 256
    n_chunk = rows_per_w // CHUNK
    JPC = CHUNK // 128
    idx_rows_w = rows_per_w // 128

    mesh = plsc.VectorSubcoreMesh(core_axis_name="c", subcore_axis_name="s")

    @functools.partial(
        pl.kernel,
        mesh=mesh,
        compiler_params=pltpu.CompilerParams(use_tc_tiling_on_sc=False),
        out_type=jax.ShapeDtypeStruct((n_rows, 2 * D), jnp.float32),
        scratch_types=[
            pltpu.VMEM((idx_rows_w, 128), jnp.int32),
            pltpu.VMEM((idx_rows_w, 128), jnp.int32),
            pltpu.VMEM((CHUNK, D), jnp.float32),
            pltpu.VMEM((CHUNK, D), jnp.float32),
            pltpu.SemaphoreType.DMA,
            pltpu.SemaphoreType.DMA,
        ],
    )
    def gather_kernel(t_hbm, il_hbm, ir_hbm, o_hbm,
                      il_v, ir_v, rl_v, rr_v, sem_l, sem_r):
        wid = lax.axis_index("s") * NC + lax.axis_index("c")
        pltpu.sync_copy(il_hbm.at[pl.ds(wid * idx_rows_w, idx_rows_w)], il_v)
        pltpu.sync_copy(ir_hbm.at[pl.ds(wid * idx_rows_w, idx_rows_w)], ir_v)

        def chunk_body(ci, carry):
            cps = []
            for j in range(JPC):
                row = ci * JPC + j
                cps.append(pltpu.async_copy(
                    t_hbm.at[il_v.at[row]], rl_v.at[pl.ds(j * 128, 128)],
                    sem_l))
                cps.append(pltpu.async_copy(
                    t_hbm.at[ir_v.at[row]], rr_v.at[pl.ds(j * 128, 128)],
                    sem_r))
            for cp in cps:
                cp.wait()
            base = wid * rows_per_w + ci * CHUNK
            pltpu.sync_copy(rl_v,
                            o_hbm.at[pl.ds(base, CHUNK), pl.ds(0, D)])
            pltpu.sync_copy(rr_v,
                            o_hbm.at[pl.ds(base, CHUNK), pl.ds(D, D)])
            return carry

        lax.fori_loop(0, n_chunk, chunk_body, 0)

    return gather_kernel(table, il, ir)


def _mlp(f, w1, b1, w2, b2):
    n_rows, D2 = f.shape
    H = w1.shape[1]
    O = w2.shape[1]
    TM = 2048
    grid = (n_rows // TM,)

    def body(f_ref, w1_ref, b1_ref, w2_ref, b2_ref, out_ref):
        h = jnp.dot(f_ref[...], w1_ref[...],
                    preferred_element_type=jnp.float32)
        h = jnp.maximum(h + b1_ref[...], 0.0)
        out_ref[...] = jnp.dot(h, w2_ref[...],
                               preferred_element_type=jnp.float32) + b2_ref[...]

    return pl.pallas_call(
        body,
        grid=grid,
        in_specs=[
            pl.BlockSpec((TM, D2), lambda i: (i, 0)),
            pl.BlockSpec((D2, H), lambda i: (0, 0)),
            pl.BlockSpec((1, H), lambda i: (0, 0)),
            pl.BlockSpec((H, O), lambda i: (0, 0)),
            pl.BlockSpec((1, O), lambda i: (0, 0)),
        ],
        out_specs=pl.BlockSpec((TM, O), lambda i: (i, 0)),
        out_shape=jax.ShapeDtypeStruct((n_rows, O), jnp.float32),
    )(f, w1, b1, w2, b2)


def kernel(soutput_f_l, soutput_f_r, matches, non_matches, start_idx,
           num_points, W1, b1, W2, b2):
    B, M, _ = matches.shape
    NM = non_matches.shape[1]
    R, D = soutput_f_l.shape
    n_static = R // B

    def global_idx(x):
        x = jnp.where(x > -1, x, num_points[:, None])
        x = jnp.clip(x, 0, n_static - 1)
        return x + start_idx[:, None]

    il = 2 * jnp.concatenate(
        [global_idx(matches[:, :, 0]), global_idx(non_matches[:, :, 0])],
        axis=0)
    ir = 2 * jnp.concatenate(
        [global_idx(matches[:, :, 1]), global_idx(non_matches[:, :, 1])],
        axis=0) + 1
    n_rows = B * (M + NM)
    il = il.reshape(n_rows // 128, 128)
    ir = ir.reshape(n_rows // 128, 128)

    table = jnp.concatenate([soutput_f_l, soutput_f_r],
                            axis=1).reshape(2 * R, D)
    f = _fused_gather(table, il, ir)

    prediction = _mlp(f, W1, b1.reshape(1, -1), W2, b2.reshape(1, -1))
    label = jnp.concatenate(
        [jnp.ones((B * M,), jnp.float32), jnp.zeros((B * NM,), jnp.float32)])
    return (prediction, label)


# speedup vs baseline: 6.0728x; 1.2602x over previous
import functools

import jax
import jax.numpy as jnp
from jax import lax
from jax.experimental import pallas as pl
from jax.experimental.pallas import tpu as pltpu
from jax.experimental.pallas import tpu_sc as plsc


def _fused_gather(table, il, ir):
    n_rows = il.shape[0] * 128
    D = table.shape[1]
    info = plsc.get_sparse_core_info()
    NC, NS = info.num_cores, info.num_subcores
    NW = NC * NS
    rows_per_w = n_rows // NW
    CHUNK = 256
    n_chunk = rows_per_w // CHUNK
    JPC = CHUNK // 128
    idx_rows_w = rows_per_w // 128

    mesh = plsc.VectorSubcoreMesh(core_axis_name="c", subcore_axis_name="s")

    @functools.partial(
        pl.kernel,
        mesh=mesh,
        compiler_params=pltpu.CompilerParams(use_tc_tiling_on_sc=False),
        out_type=jax.ShapeDtypeStruct((n_rows, 2 * D), jnp.float32),
        scratch_types=[
            pltpu.VMEM((idx_rows_w, 128), jnp.int32),
            pltpu.VMEM((idx_rows_w, 128), jnp.int32),
            pltpu.VMEM((CHUNK, D), jnp.float32),
            pltpu.VMEM((CHUNK, D), jnp.float32),
            pltpu.SemaphoreType.DMA,
            pltpu.SemaphoreType.DMA,
        ],
    )
    def gather_kernel(t_hbm, il_hbm, ir_hbm, o_hbm,
                      il_v, ir_v, rl_v, rr_v, sem_l, sem_r):
        wid = lax.axis_index("s") * NC + lax.axis_index("c")
        pltpu.sync_copy(il_hbm.at[pl.ds(wid * idx_rows_w, idx_rows_w)], il_v)
        pltpu.sync_copy(ir_hbm.at[pl.ds(wid * idx_rows_w, idx_rows_w)], ir_v)

        def chunk_body(ci, carry):
            cps = []
            for j in range(JPC):
                row = ci * JPC + j
                cps.append(pltpu.async_copy(
                    t_hbm.at[il_v.at[row]], rl_v.at[pl.ds(j * 128, 128)],
                    sem_l))
                cps.append(pltpu.async_copy(
                    t_hbm.at[ir_v.at[row]], rr_v.at[pl.ds(j * 128, 128)],
                    sem_r))
            for cp in cps:
                cp.wait()
            base = wid * rows_per_w + ci * CHUNK
            pltpu.sync_copy(rl_v,
                            o_hbm.at[pl.ds(base, CHUNK), pl.ds(0, D)])
            pltpu.sync_copy(rr_v,
                            o_hbm.at[pl.ds(base, CHUNK), pl.ds(D, D)])
            return carry

        lax.fori_loop(0, n_chunk, chunk_body, 0)

    return gather_kernel(table, il, ir)


def _mlp(f, w1, b1, w2, b2):
    n_rows, D2 = f.shape
    H = w1.shape[1]
    O = w2.shape[0]
    TM = 2048
    grid = (n_rows // TM,)

    def body(f_ref, w1_ref, b1_ref, w2t_ref, b2_ref, out_ref):
        h = jnp.dot(f_ref[...], w1_ref[...],
                    preferred_element_type=jnp.float32)
        h = jnp.maximum(h + b1_ref[...], 0.0)
        pt = jax.lax.dot_general(
            w2t_ref[...], h, (((1,), (1,)), ((), ())),
            preferred_element_type=jnp.float32)
        out_ref[...] = pt + b2_ref[...]

    return pl.pallas_call(
        body,
        grid=grid,
        in_specs=[
            pl.BlockSpec((TM, D2), lambda i: (i, 0)),
            pl.BlockSpec((D2, H), lambda i: (0, 0)),
            pl.BlockSpec((1, H), lambda i: (0, 0)),
            pl.BlockSpec((O, H), lambda i: (0, 0)),
            pl.BlockSpec((O, 1), lambda i: (0, 0)),
        ],
        out_specs=pl.BlockSpec((O, TM), lambda i: (0, i)),
        out_shape=jax.ShapeDtypeStruct((O, n_rows), jnp.float32),
    )(f, w1, b1, w2, b2)


def kernel(soutput_f_l, soutput_f_r, matches, non_matches, start_idx,
           num_points, W1, b1, W2, b2):
    B, M, _ = matches.shape
    NM = non_matches.shape[1]
    R, D = soutput_f_l.shape
    n_static = R // B

    def global_idx(x):
        x = jnp.where(x > -1, x, num_points[:, None])
        x = jnp.clip(x, 0, n_static - 1)
        return x + start_idx[:, None]

    il = 2 * jnp.concatenate(
        [global_idx(matches[:, :, 0]), global_idx(non_matches[:, :, 0])],
        axis=0)
    ir = 2 * jnp.concatenate(
        [global_idx(matches[:, :, 1]), global_idx(non_matches[:, :, 1])],
        axis=0) + 1
    n_rows = B * (M + NM)
    il = il.reshape(n_rows // 128, 128)
    ir = ir.reshape(n_rows // 128, 128)

    table = jnp.concatenate([soutput_f_l, soutput_f_r],
                            axis=1).reshape(2 * R, D)
    f = _fused_gather(table, il, ir)

    prediction = _mlp(f, W1, b1.reshape(1, -1), W2.T, b2.reshape(-1, 1)).T
    label = jnp.concatenate(
        [jnp.ones((B * M,), jnp.float32), jnp.zeros((B * NM,), jnp.float32)])
    return (prediction, label)


# speedup vs baseline: 6.2590x; 1.0307x over previous
import functools

import jax
import jax.numpy as jnp
from jax import lax
from jax.experimental import pallas as pl
from jax.experimental.pallas import tpu as pltpu
from jax.experimental.pallas import tpu_sc as plsc


def _fused_gather(table, il, ir):
    n_rows = il.shape[0] * 128
    D = table.shape[1]
    info = plsc.get_sparse_core_info()
    NC, NS = info.num_cores, info.num_subcores
    NW = NC * NS
    rows_per_w = n_rows // NW
    CHUNK = 256
    n_chunk = rows_per_w // CHUNK
    JPC = CHUNK // 128
    idx_rows_w = rows_per_w // 128

    mesh = plsc.VectorSubcoreMesh(core_axis_name="c", subcore_axis_name="s")

    @functools.partial(
        pl.kernel,
        mesh=mesh,
        compiler_params=pltpu.CompilerParams(use_tc_tiling_on_sc=False),
        out_type=jax.ShapeDtypeStruct((n_rows, 2 * D), jnp.float32),
        scratch_types=[
            pltpu.VMEM((idx_rows_w, 128), jnp.int32),
            pltpu.VMEM((idx_rows_w, 128), jnp.int32),
            pltpu.VMEM((CHUNK, D), jnp.float32),
            pltpu.VMEM((CHUNK, D), jnp.float32),
            pltpu.SemaphoreType.DMA,
            pltpu.SemaphoreType.DMA,
        ],
    )
    def gather_kernel(t_hbm, il_hbm, ir_hbm, o_hbm,
                      il_v, ir_v, rl_v, rr_v, sem_l, sem_r):
        wid = lax.axis_index("s") * NC + lax.axis_index("c")
        pltpu.sync_copy(il_hbm.at[pl.ds(wid * idx_rows_w, idx_rows_w)], il_v)
        pltpu.sync_copy(ir_hbm.at[pl.ds(wid * idx_rows_w, idx_rows_w)], ir_v)

        def chunk_body(ci, carry):
            cps = []
            for j in range(JPC):
                row = ci * JPC + j
                cps.append(pltpu.async_copy(
                    t_hbm.at[il_v.at[row]], rl_v.at[pl.ds(j * 128, 128)],
                    sem_l))
                cps.append(pltpu.async_copy(
                    t_hbm.at[ir_v.at[row]], rr_v.at[pl.ds(j * 128, 128)],
                    sem_r))
            for cp in cps:
                cp.wait()
            base = wid * rows_per_w + ci * CHUNK
            pltpu.sync_copy(rl_v,
                            o_hbm.at[pl.ds(base, CHUNK), pl.ds(0, D)])
            pltpu.sync_copy(rr_v,
                            o_hbm.at[pl.ds(base, CHUNK), pl.ds(D, D)])
            return carry

        lax.fori_loop(0, n_chunk, chunk_body, 0)

    return gather_kernel(table, il, ir)


def _mlp(f, w1, b1, w2, b2):
    n_rows, D2 = f.shape
    H = w1.shape[1]
    O = w2.shape[0]
    TM = 2048
    grid = (n_rows // TM,)

    def body(f_ref, w1_ref, b1_ref, w2t_ref, b2_ref, out_ref):
        h = jnp.dot(f_ref[...], w1_ref[...],
                    preferred_element_type=jnp.float32)
        h = jnp.maximum(h + b1_ref[...], 0.0)
        pt = jax.lax.dot_general(
            w2t_ref[...], h, (((1,), (1,)), ((), ())),
            preferred_element_type=jnp.float32)
        out_ref[...] = pt + b2_ref[...]

    return pl.pallas_call(
        body,
        grid=grid,
        in_specs=[
            pl.BlockSpec((TM, D2), lambda i: (i, 0)),
            pl.BlockSpec((D2, H), lambda i: (0, 0)),
            pl.BlockSpec((1, H), lambda i: (0, 0)),
            pl.BlockSpec((O, H), lambda i: (0, 0)),
            pl.BlockSpec((O, 1), lambda i: (0, 0)),
        ],
        out_specs=pl.BlockSpec((O, TM), lambda i: (0, i)),
        out_shape=jax.ShapeDtypeStruct((O, n_rows), jnp.float32),
    )(f, w1, b1, w2, b2)


def kernel(soutput_f_l, soutput_f_r, matches, non_matches, start_idx,
           num_points, W1, b1, W2, b2):
    B, M, _ = matches.shape
    NM = non_matches.shape[1]
    R, D = soutput_f_l.shape
    n_static = R // B

    def global_idx(x):
        x = jnp.where(x > -1, x, num_points[:, None])
        x = jnp.clip(x, 0, n_static - 1)
        return x + start_idx[:, None]

    il = 2 * jnp.concatenate(
        [global_idx(matches[:, :, 0]), global_idx(non_matches[:, :, 0])],
        axis=0)
    ir = 2 * jnp.concatenate(
        [global_idx(matches[:, :, 1]), global_idx(non_matches[:, :, 1])],
        axis=0) + 1
    n_rows = B * (M + NM)
    il = il.reshape(n_rows // 128, 128)
    ir = ir.reshape(n_rows // 128, 128)

    table = jnp.concatenate([soutput_f_l, soutput_f_r],
                            axis=1).reshape(2 * R, D)

    NSPLIT = 2
    idx_rows = n_rows // 128
    srows = idx_rows // NSPLIT
    preds = []
    for s in range(NSPLIT):
        il_s = jax.lax.slice_in_dim(il, s * srows, (s + 1) * srows, axis=0)
        ir_s = jax.lax.slice_in_dim(ir, s * srows, (s + 1) * srows, axis=0)
        f_s = _fused_gather(table, il_s, ir_s)
        preds.append(_mlp(f_s, W1, b1.reshape(1, -1), W2.T,
                          b2.reshape(-1, 1)))
    prediction = jnp.concatenate(preds, axis=1).T
    label = jnp.concatenate(
        [jnp.ones((B * M,), jnp.float32), jnp.zeros((B * NM,), jnp.float32)])
    return (prediction, label)
